# baseline (speedup 1.0000x reference)
import jax
import jax.numpy as jnp
from jax import lax
from jax.experimental import pallas as pl
from jax.experimental.pallas import tpu as pltpu
from jax.experimental.pallas import tpu_sc as plsc

_BATCH = 16384
_SEQ = 200
_KMERS = 1024
_NW = 32
_UR = 8
_UC = 512
_NCOLB = _BATCH // _UC
_UNITS = (_SEQ // _UR) * _NCOLB
_KMAIN = _UNITS // _NW
_NBUF = 3
_LANES = 16


def _gather_kernel(idx_hbm, table_hbm, out_hbm, tab_v, idx_v, out_v,
                   in_sems, out_sems):
    wid = lax.axis_index("s") * 2 + lax.axis_index("c")

    def _unit_slices(u):
        r0 = (u // _NCOLB) * _UR
        c0 = (u % _NCOLB) * _UC
        return pl.ds(r0, _UR), pl.ds(c0, _UC)

    def _start_in(k, off, sem):
        rs, cs = _unit_slices(wid + k * _NW)
        pltpu.async_copy(idx_hbm.at[rs, cs], idx_v.at[pl.ds(off, _UR)], sem)

    for bb in range(_NBUF):
        _start_in(bb, bb * _UR, in_sems.at[bb])

    pltpu.sync_copy(table_hbm, tab_v)

    def _exp_body(i):
        sl = pl.ds(i * _LANES, _LANES)
        tab_v[sl] = jnp.exp(tab_v[sl])

    pl.loop(0, _KMERS // _LANES)(_exp_body)

    def _round(k):
        b = k % _NBUF
        off = b * _UR
        rs, cs = _unit_slices(wid + k * _NW)
        pltpu.make_async_copy(idx_hbm.at[rs, cs],
                              idx_v.at[pl.ds(off, _UR)],
                              in_sems.at[b]).wait()

        @pl.when(k >= _NBUF)
        def _wait_prev_store():
            rp, cp = _unit_slices(wid + (k - _NBUF) * _NW)
            pltpu.make_async_copy(out_v.at[pl.ds(off, _UR)],
                                  out_hbm.at[rp, cp], out_sems.at[b]).wait()

        def _row_body(r):
            def _group_body(g):
                sl = pl.ds(g * _LANES, _LANES)
                out_v[off + r, sl] = plsc.load_gather(tab_v,
                                                      [idx_v[off + r, sl]])
            plsc.parallel_loop(0, _UC // _LANES, unroll=8)(_group_body)

        pl.loop(0, _UR)(_row_body)

        pltpu.async_copy(out_v.at[pl.ds(off, _UR)], out_hbm.at[rs, cs],
                         out_sems.at[b])

        @pl.when(k + _NBUF < _KMAIN)
        def _start_next():
            _start_in(k + _NBUF, off, in_sems.at[b])

    pl.loop(0, _KMAIN)(_round)

    for b in range(_NBUF):
        rs, cs = _unit_slices(wid + b * _NW)
        pltpu.make_async_copy(out_v.at[pl.ds(b * _UR, _UR)],
                              out_hbm.at[rs, cs], out_sems.at[b]).wait()


@jax.jit
def kernel(encoded_parents, masks, kmer_embedding):
    del masks
    idx_t = encoded_parents.T
    table_flat = kmer_embedding.reshape(_KMERS)

    mesh = plsc.VectorSubcoreMesh(core_axis_name="c", subcore_axis_name="s")
    out_t = pl.kernel(
        _gather_kernel,
        mesh=mesh,
        out_type=jax.ShapeDtypeStruct((_SEQ, _BATCH), jnp.float32),
        scratch_types=[
            pltpu.VMEM((_KMERS,), jnp.float32),
            pltpu.VMEM((_NBUF * _UR, _UC), jnp.int32),
            pltpu.VMEM((_NBUF * _UR, _UC), jnp.float32),
            pltpu.SemaphoreType.DMA((_NBUF,)),
            pltpu.SemaphoreType.DMA((_NBUF,)),
        ],
        compiler_params=pltpu.CompilerParams(needs_layout_passes=False,
                                             use_tc_tiling_on_sc=True),
    )(idx_t, table_flat)
    return out_t.T


# speedup vs baseline: 1.0586x; 1.0586x over previous
import jax
import jax.numpy as jnp
from jax import lax
from jax.experimental import pallas as pl
from jax.experimental.pallas import tpu as pltpu
from jax.experimental.pallas import tpu_sc as plsc

_BATCH = 16384
_SEQ = 200
_KMERS = 1024
_NW = 32
_UR = 8
_UC = 1024
_NCOLB = _BATCH // _UC
_UNITS = (_SEQ // _UR) * _NCOLB
_KMAIN = _UNITS // _NW
_NEXTRA = _UNITS - _KMAIN * _NW
_NBUF = 3
_GROUPS = _UR * _UC // 16
_GPR = _UC // 16
_LANES = 16


def _gather_kernel(idx_hbm, table_hbm, out_hbm, tab_v, idx_v, out_v,
                   in_sems, out_sems):
    wid = lax.axis_index("s") * 2 + lax.axis_index("c")
    nk = jnp.where(wid < _NEXTRA, _KMAIN + 1, _KMAIN)

    def _unit_slices(u):
        r0 = (u // _NCOLB) * _UR
        c0 = (u % _NCOLB) * _UC
        return pl.ds(r0, _UR), pl.ds(c0, _UC)

    def _start_in(k, off, sem):
        rs, cs = _unit_slices(wid + k * _NW)
        pltpu.async_copy(idx_hbm.at[rs, cs], idx_v.at[pl.ds(off, _UR)], sem)

    for bb in range(_NBUF):
        _start_in(bb, bb * _UR, in_sems.at[bb])

    pltpu.sync_copy(table_hbm, tab_v)

    def _exp_body(i):
        sl = pl.ds(i * _LANES, _LANES)
        tab_v[sl] = jnp.exp(tab_v[sl])

    pl.loop(0, _KMERS // _LANES)(_exp_body)

    def _round(k):
        b = k % _NBUF
        off = b * _UR
        rs, cs = _unit_slices(wid + k * _NW)
        pltpu.make_async_copy(idx_hbm.at[rs, cs],
                              idx_v.at[pl.ds(off, _UR)],
                              in_sems.at[b]).wait()

        @pl.when(k >= _NBUF)
        def _wait_prev_store():
            rp, cp = _unit_slices(wid + (k - _NBUF) * _NW)
            pltpu.make_async_copy(out_v.at[pl.ds(off, _UR)],
                                  out_hbm.at[rp, cp], out_sems.at[b]).wait()

        def _group_body(g):
            r = off + (g // _GPR)
            sl = pl.ds((g % _GPR) * _LANES, _LANES)
            out_v[r, sl] = plsc.load_gather(tab_v, [idx_v[r, sl]])

        plsc.parallel_loop(0, _GROUPS, unroll=8)(_group_body)

        pltpu.async_copy(out_v.at[pl.ds(off, _UR)], out_hbm.at[rs, cs],
                         out_sems.at[b])

        @pl.when(k + _NBUF < nk)
        def _start_next():
            _start_in(k + _NBUF, off, in_sems.at[b])

    pl.loop(0, nk)(_round)

    for b in range(_NBUF):
        rs, cs = _unit_slices(wid + b * _NW)
        pltpu.make_async_copy(out_v.at[pl.ds(b * _UR, _UR)],
                              out_hbm.at[rs, cs], out_sems.at[b]).wait()


@jax.jit
def kernel(encoded_parents, masks, kmer_embedding):
    del masks
    idx_t = encoded_parents.T
    table_flat = kmer_embedding.reshape(_KMERS)

    mesh = plsc.VectorSubcoreMesh(core_axis_name="c", subcore_axis_name="s")
    out_t = pl.kernel(
        _gather_kernel,
        mesh=mesh,
        out_type=jax.ShapeDtypeStruct((_SEQ, _BATCH), jnp.float32),
        scratch_types=[
            pltpu.VMEM((_KMERS,), jnp.float32),
            pltpu.VMEM((_NBUF * _UR, _UC), jnp.int32),
            pltpu.VMEM((_NBUF * _UR, _UC), jnp.float32),
            pltpu.SemaphoreType.DMA((_NBUF,)),
            pltpu.SemaphoreType.DMA((_NBUF,)),
        ],
        compiler_params=pltpu.CompilerParams(needs_layout_passes=False,
                                             use_tc_tiling_on_sc=True),
    )(idx_t, table_flat)
    return out_t.T
